# baseline (speedup 1.0000x reference)
import functools

import jax
import jax.numpy as jnp
import numpy as np
from jax import lax
from jax.experimental import pallas as pl
from jax.experimental.pallas import tpu as pltpu
from jax.experimental.pallas import tpu_sc as plsc

B, N, H, HIN, K, OUT, NPE, NRBF, NLAYERS = 2, 512, 128, 128, 30, 400, 16, 16, 3
SCALE = 30.0
NK = N * K
ENK = B * NK
BN = B * N

def _dot(a, b):
    return jnp.dot(a, b, preferred_element_type=jnp.float32)


_PAR1 = pltpu.CompilerParams(dimension_semantics=("parallel",))


def _ln(x, g, b):
    mu = jnp.mean(x, axis=-1, keepdims=True)
    var = jnp.mean((x - mu) ** 2, axis=-1, keepdims=True)
    return (x - mu) / jnp.sqrt(var + 1e-5) * g + b


def _topk_body(xcol_ref, xrow_ref, chain_ref, eidx_ref, dsel_ref, offs_ref,
               partner_ref):
    xc = xcol_ref[0]
    xr = xrow_ref[0]
    ch = chain_ref[0]
    dx = xc[:, 0:1] - xr[0:1, :]
    dy = xc[:, 1:2] - xr[1:2, :]
    dz = xc[:, 2:3] - xr[2:3, :]
    s = (dx * dx + dy * dy) + dz * dz
    d = jnp.sqrt(s + 1e-6)
    lane = lax.broadcasted_iota(jnp.int32, (N, N), 1)
    cur = d
    idx_cols = []
    val_cols = []
    for _ in range(K):
        mn = jnp.min(cur, axis=1, keepdims=True)
        am = jnp.min(jnp.where(cur == mn, lane, N), axis=1, keepdims=True)
        idx_cols.append(am)
        val_cols.append(mn)
        cur = jnp.where(lane == am, jnp.float32(np.inf), cur)
    eidx = jnp.concatenate(idx_cols, axis=1)
    dsel = jnp.concatenate(val_cols, axis=1)
    t1 = jnp.sum((ch < 1).astype(jnp.int32), axis=1, keepdims=True)
    t2 = jnp.sum((ch < 2).astype(jnp.int32), axis=1, keepdims=True)
    t3 = jnp.sum((ch < 3).astype(jnp.int32), axis=1, keepdims=True)

    def chain_of(v):
        return ((v >= t1).astype(jnp.int32) + (v >= t2).astype(jnp.int32)
                + (v >= t3).astype(jnp.int32))

    i_col = lax.broadcasted_iota(jnp.int32, (N, 1), 0)
    same = (chain_of(eidx) == chain_of(i_col)).astype(jnp.float32)
    offs = (eidx - i_col).astype(jnp.float32) * same
    big = jnp.int32(9999)
    w = jnp.full((N, N), big, jnp.int32)
    for k in range(K):
        w = jnp.where(lane == idx_cols[k], jnp.int32(k), w)
    wt = w.T
    kp_cols = []
    for k in range(K):
        kp_cols.append(jnp.min(jnp.where(lane == idx_cols[k], wt, big),
                               axis=1, keepdims=True))
    kp = jnp.concatenate(kp_cols, axis=1)
    found = kp < big
    k_row = lax.broadcasted_iota(jnp.int32, (N, K), 1)
    b_off = pl.program_id(0) * NK
    partner = jnp.where(found, eidx * K + kp, i_col * K + k_row) + b_off
    eidx_ref[0] = eidx
    dsel_ref[0] = dsel
    offs_ref[0] = offs
    partner_ref[0] = partner


def _run_topk(xcol, xrow, chain3):
    return pl.pallas_call(
        _topk_body,
        grid=(B,),
        in_specs=[
            pl.BlockSpec((1, N, 8), lambda b: (b, 0, 0)),
            pl.BlockSpec((1, 8, N), lambda b: (b, 0, 0)),
            pl.BlockSpec((1, 1, N), lambda b: (b, 0, 0)),
        ],
        out_specs=[
            pl.BlockSpec((1, N, K), lambda b: (b, 0, 0)),
            pl.BlockSpec((1, N, K), lambda b: (b, 0, 0)),
            pl.BlockSpec((1, N, K), lambda b: (b, 0, 0)),
            pl.BlockSpec((1, N, K), lambda b: (b, 0, 0)),
        ],
        out_shape=[
            jax.ShapeDtypeStruct((B, N, K), jnp.int32),
            jax.ShapeDtypeStruct((B, N, K), jnp.float32),
            jax.ShapeDtypeStruct((B, N, K), jnp.float32),
            jax.ShapeDtypeStruct((B, N, K), jnp.int32),
        ],
        compiler_params=_PAR1,
    )(xcol, xrow, chain3)


def _dih_body(xb_ref, out_ref):
    xb = xb_ref[0]
    x = xb[0:1]
    y = xb[1:2]
    z = xb[2:3]

    def sh(a, s):
        return pltpu.roll(a, 3 * N - s, 1)

    def unit(ax, ay, az):
        s = (ax * ax + ay * ay) + az * az
        nrm = jnp.sqrt(s + 1e-12)
        return ax / nrm, ay / nrm, az / nrm

    def cross(ax, ay, az, bx, by, bz):
        return (ay * bz - az * by, az * bx - ax * bz, ax * by - ay * bx)

    dx, dy, dz = sh(x, 1) - x, sh(y, 1) - y, sh(z, 1) - z
    ux, uy, uz = unit(dx, dy, dz)
    vx, vy, vz = sh(ux, 1), sh(uy, 1), sh(uz, 1)
    wx, wy, wz = sh(ux, 2), sh(uy, 2), sh(uz, 2)
    ax_, ay_, az_ = cross(ux, uy, uz, vx, vy, vz)
    bx_, by_, bz_ = cross(vx, vy, vz, wx, wy, wz)
    n2x, n2y, n2z = unit(ax_, ay_, az_)
    n1x, n1y, n1z = unit(bx_, by_, bz_)
    cosd = (n2x * n1x + n2y * n1y) + n2z * n1z
    cosd = jnp.clip(cosd, -1.0 + 1e-7, 1.0 - 1e-7)
    sgn = jnp.sign((ux * n1x + uy * n1y) + uz * n1z)
    sind = sgn * jnp.sqrt(1.0 - cosd * cosd)
    cosf = pltpu.roll(cosd, 1, 1)
    sinf = pltpu.roll(sind, 1, 1)
    lt = lax.broadcasted_iota(jnp.int32, (1, 3 * N), 1)
    valid = (lt >= 1) & (lt <= 3 * N - 3)
    cosf = jnp.where(valid, cosf, 1.0)
    sinf = jnp.where(valid, sinf, 0.0)
    out_ref[0] = jnp.concatenate(
        [cosf, sinf, jnp.zeros((6, 3 * N), jnp.float32)], axis=0)


def _run_dih(xb8):
    return pl.pallas_call(
        _dih_body,
        grid=(B,),
        in_specs=[pl.BlockSpec((1, 8, 3 * N), lambda b: (b, 0, 0))],
        out_specs=pl.BlockSpec((1, 8, 3 * N), lambda b: (b, 0, 0)),
        out_shape=jax.ShapeDtypeStruct((B, 8, 3 * N), jnp.float32),
        compiler_params=_PAR1,
    )(xb8)


def _nodefeat_body(dih_ref, vemb_ref, fw_ref, fb_ref, g_ref, gb_ref,
                   wv1_ref, wv2_ref, bv_ref, out_ref):
    v = _ln(_dot(dih_ref[...], fw_ref[...]) + fb_ref[...],
            g_ref[...], gb_ref[...])
    out_ref[...] = (_dot(v, wv1_ref[...]) + _dot(vemb_ref[...], wv2_ref[...])
                    + bv_ref[...])


def _run_nodefeat(dih8, vemb, fw8, fb, g, gb, wv1, wv2, bv):
    return pl.pallas_call(
        _nodefeat_body,
        out_shape=jax.ShapeDtypeStruct((BN, H), jnp.float32),
    )(dih8, vemb, fw8, fb, g, gb, wv1, wv2, bv)


_RT2 = 2048

_SIGMA = 20.0 / NRBF


def _edgefeat_body(dsel_ref, offs_ref, eg_ref,
                   few_ref, feb_ref, g_ref, gb_ref,
                   we1_ref, we2_ref, be_ref, he_ref):
    dsel = dsel_ref[...]
    offs = offs_ref[...]
    io8 = lax.broadcasted_iota(jnp.int32, (1, NPE // 2), 1).astype(jnp.float32)
    freq = jnp.exp(io8 * jnp.float32(2.0 * (-np.log(10000.0) / NPE)))
    io16 = lax.broadcasted_iota(jnp.int32, (1, NRBF), 1).astype(jnp.float32)
    mu = 2.0 + io16 * jnp.float32(20.0 / (NRBF - 1))
    ang = offs * freq
    epos = jnp.concatenate([jnp.cos(ang), jnp.sin(ang)], axis=1)
    rbf = jnp.exp(-(((dsel - mu) / _SIGMA) ** 2))
    f = jnp.concatenate([epos, rbf], axis=1)
    e1 = _ln(_dot(f, few_ref[...]) + feb_ref[...], g_ref[...], gb_ref[...])
    he_ref[...] = (_dot(e1, we1_ref[...]) + _dot(eg_ref[...], we2_ref[...])
                   + be_ref[...])


def _run_edgefeat(dsel_f, offs_f, eg, few, feb, g, gb, we1, we2, be):
    nblk = ENK // _RT2
    full = lambda a: pl.BlockSpec(a.shape, lambda i: (0,) * a.ndim)
    return pl.pallas_call(
        _edgefeat_body,
        grid=(nblk,),
        in_specs=[
            pl.BlockSpec((_RT2, 1), lambda i: (i, 0)),
            pl.BlockSpec((_RT2, 1), lambda i: (i, 0)),
            pl.BlockSpec((_RT2, HIN), lambda i: (i, 0)),
            full(few), full(feb), full(g), full(gb),
            full(we1), full(we2), full(be),
        ],
        out_specs=pl.BlockSpec((_RT2, H), lambda i: (i, 0)),
        out_shape=jax.ShapeDtypeStruct((ENK, H), jnp.float32),
        compiler_params=_PAR1,
    )(dsel_f, offs_f, eg, few, feb, g, gb, we1, we2, be)


_NB = 128
_RE = _NB * K
_NGRID = BN // _NB


def _gelu(x):
    return jax.nn.gelu(x)


def _edge_layer_body(he_ref, hep_ref, hvj_ref,
                     w1a_ref, w1b_ref, w1c_ref, b1_ref,
                     w2_ref, b2_ref, w3_ref, b3_ref, g_ref, gb_ref, out_ref):
    hm = 0.5 * (he_ref[...] + hep_ref[...])
    hvj3 = hvj_ref[...].reshape(_NB, K, H)
    xa = _dot(hvj3[:, 0, :], w1a_ref[...])
    xa = jnp.broadcast_to(xa[:, None, :], (_NB, K, H)).reshape(_RE, H)
    x = (xa + _dot(hvj_ref[...], w1b_ref[...])
         + _dot(hm, w1c_ref[...]) + b1_ref[...])
    x = _gelu(x)
    x = _gelu(_dot(x, w2_ref[...]) + b2_ref[...])
    m = _dot(x, w3_ref[...]) + b3_ref[...]
    out_ref[...] = _ln(hm + m, g_ref[...], gb_ref[...])


def _run_edge_layer(he, hep, hvj, p):
    full = lambda a: pl.BlockSpec(a.shape, lambda i: (0,) * a.ndim)
    erow = pl.BlockSpec((_RE, H), lambda i: (i, 0))
    return pl.pallas_call(
        _edge_layer_body,
        grid=(_NGRID,),
        in_specs=[erow, erow, erow] + [full(a) for a in p],
        out_specs=erow,
        out_shape=jax.ShapeDtypeStruct((ENK, H), jnp.float32),
        compiler_params=_PAR1,
    )(he, hep, hvj, *p)


def _node_layer_body(hv_ref, hvj_ref, he_ref, hep_ref,
                     w1a_ref, w1b_ref, w1c_ref, b1_ref,
                     w2_ref, b2_ref, w3_ref, b3_ref,
                     g1_ref, g1b_ref, fi_ref, fib_ref, fo_ref, fob_ref,
                     g2_ref, g2b_ref, hv_out_ref, hm_out_ref):
    hm = 0.5 * (he_ref[...] + hep_ref[...])
    hm_out_ref[...] = hm
    hv = hv_ref[...]
    xa = _dot(hv, w1a_ref[...])
    xa = jnp.broadcast_to(xa[:, None, :], (_NB, K, H)).reshape(_RE, H)
    x = (xa + _dot(hvj_ref[...], w1b_ref[...])
         + _dot(hm, w1c_ref[...]) + b1_ref[...])
    x = _gelu(x)
    x = _gelu(_dot(x, w2_ref[...]) + b2_ref[...])
    m = _dot(x, w3_ref[...]) + b3_ref[...]
    dh = jnp.sum(m.reshape(_NB, K, H), axis=1) / SCALE
    hv1 = _ln(hv + dh, g1_ref[...], g1b_ref[...])
    ff = _dot(_gelu(_dot(hv1, fi_ref[...]) + fib_ref[...]), fo_ref[...]) \
        + fob_ref[...]
    hv_out_ref[...] = _ln(hv1 + ff, g2_ref[...], g2b_ref[...])


def _node_layer_last_body(hv_ref, hvj_ref, he_ref, hep_ref,
                          w1a_ref, w1b_ref, w1c_ref, b1_ref,
                          w2_ref, b2_ref, w3_ref, b3_ref,
                          g1_ref, g1b_ref, fi_ref, fib_ref, fo_ref, fob_ref,
                          g2_ref, g2b_ref, wo_ref, bo_ref,
                          hv_out_ref, etab_ref):
    hm = 0.5 * (he_ref[...] + hep_ref[...])
    etab_ref[...] = _dot(hm, wo_ref[...]) + bo_ref[...]
    hv = hv_ref[...]
    xa = _dot(hv, w1a_ref[...])
    xa = jnp.broadcast_to(xa[:, None, :], (_NB, K, H)).reshape(_RE, H)
    x = (xa + _dot(hvj_ref[...], w1b_ref[...])
         + _dot(hm, w1c_ref[...]) + b1_ref[...])
    x = _gelu(x)
    x = _gelu(_dot(x, w2_ref[...]) + b2_ref[...])
    m = _dot(x, w3_ref[...]) + b3_ref[...]
    dh = jnp.sum(m.reshape(_NB, K, H), axis=1) / SCALE
    hv1 = _ln(hv + dh, g1_ref[...], g1b_ref[...])
    ff = _dot(_gelu(_dot(hv1, fi_ref[...]) + fib_ref[...]), fo_ref[...]) \
        + fob_ref[...]
    hv_out_ref[...] = _ln(hv1 + ff, g2_ref[...], g2b_ref[...])


def _run_node_layer_last(hv, hvj, he, hep, p):
    full = lambda a: pl.BlockSpec(a.shape, lambda i: (0,) * a.ndim)
    erow = pl.BlockSpec((_RE, H), lambda i: (i, 0))
    nrow = pl.BlockSpec((_NB, H), lambda i: (i, 0))
    return pl.pallas_call(
        _node_layer_last_body,
        grid=(_NGRID,),
        in_specs=[nrow, erow, erow, erow] + [full(a) for a in p],
        out_specs=[nrow, pl.BlockSpec((_RE, OUT), lambda i: (i, 0))],
        out_shape=[
            jax.ShapeDtypeStruct((BN, H), jnp.float32),
            jax.ShapeDtypeStruct((ENK, OUT), jnp.float32),
        ],
        compiler_params=_PAR1,
    )(hv, hvj, he, hep, *p)


def _run_node_layer(hv, hvj, he, hep, p):
    full = lambda a: pl.BlockSpec(a.shape, lambda i: (0,) * a.ndim)
    erow = pl.BlockSpec((_RE, H), lambda i: (i, 0))
    nrow = pl.BlockSpec((_NB, H), lambda i: (i, 0))
    return pl.pallas_call(
        _node_layer_body,
        grid=(_NGRID,),
        in_specs=[nrow, erow, erow, erow] + [full(a) for a in p],
        out_specs=[nrow, erow],
        out_shape=[
            jax.ShapeDtypeStruct((BN, H), jnp.float32),
            jax.ShapeDtypeStruct((ENK, H), jnp.float32),
        ],
        compiler_params=_PAR1,
    )(hv, hvj, he, hep, *p)


_RT5 = 1920


def _etab_body(he_ref, w_ref, b_ref, out_ref):
    out_ref[...] = _dot(he_ref[...], w_ref[...]) + b_ref[...]


def _run_etab(he, w, b):
    return pl.pallas_call(
        _etab_body,
        grid=(ENK // _RT5,),
        in_specs=[
            pl.BlockSpec((_RT5, H), lambda i: (i, 0)),
            pl.BlockSpec((H, OUT), lambda i: (0, 0)),
            pl.BlockSpec((1, OUT), lambda i: (0, 0)),
        ],
        out_specs=pl.BlockSpec((_RT5, OUT), lambda i: (i, 0)),
        out_shape=jax.ShapeDtypeStruct((ENK, OUT), jnp.float32),
    )(he, w, b)


def _sc_multi_gather(specs):
    info = plsc.get_sparse_core_info()
    nc, ns = info.num_cores, info.num_subcores
    nw = nc * ns
    mesh = plsc.VectorSubcoreMesh(core_axis_name="c", subcore_axis_name="s")
    out_type = []
    scratch = []
    meta = []
    for table, idx in specs:
        v, d = table.shape
        r = idx.shape[0]
        assert r % (8 * nw) == 0, (r, nw)
        bpw = r // nw
        chunk = bpw if bpw <= 240 else 240
        assert bpw % chunk == 0
        out_type.append(jax.ShapeDtypeStruct((r, d), table.dtype))
        for _ in range(2):
            scratch.append(pltpu.VMEM((chunk,), jnp.int32))
            scratch.append(pltpu.VMEM((chunk, d), table.dtype))
            scratch.append(pltpu.SemaphoreType.DMA)
        meta.append((bpw, chunk))

    nspec = len(specs)

    @functools.partial(pl.kernel, out_type=tuple(out_type), mesh=mesh,
                       scratch_types=tuple(scratch))
    def gather_kernel(*refs):
        tables = refs[0:2 * nspec:2]
        idxs = refs[1:2 * nspec:2]
        outs = refs[2 * nspec:3 * nspec]
        scr = refs[3 * nspec:]
        wid = lax.axis_index("s") * nc + lax.axis_index("c")
        for si in range(nspec):
            bpw, chunk = meta[si]
            sl = scr[6 * si:6 * si + 6]
            idx_v = (sl[0], sl[3])
            rows_v = (sl[1], sl[4])
            sems = (sl[2], sl[5])
            base0 = wid * bpw
            nch = bpw // chunk

            def fire(c, si=si, base0=base0, chunk=chunk):
                base = base0 + c * chunk
                pltpu.sync_copy(idxs[si].at[pl.ds(base, chunk)],
                                idx_v[c % 2])
                return pltpu.async_copy(tables[si].at[idx_v[c % 2]],
                                        rows_v[c % 2], sems[c % 2])

            pend = {0: fire(0)}
            for c in range(nch):
                if c + 1 < nch:
                    pend[c + 1] = fire(c + 1)
                pend[c].wait()
                pltpu.sync_copy(
                    rows_v[c % 2],
                    outs[si].at[pl.ds(base0 + c * chunk, chunk)])

    flat_in = []
    for table, idx in specs:
        flat_in += [table, idx]
    res = gather_kernel(*flat_in)
    if not isinstance(res, (list, tuple)):
        res = (res,)
    return list(res)


def kernel(V_embed, E_embed, X, x_mask, chain_idx, params):
    Xca = X[:, :, 1, :]
    xcol = jnp.pad(Xca, ((0, 0), (0, 0), (0, 5)))
    xrow = jnp.transpose(xcol, (0, 2, 1))
    chain3 = chain_idx.astype(jnp.int32).reshape(B, 1, N)

    eidx, dsel, offs, partner = _run_topk(xcol, xrow, chain3)
    E_idx = eidx
    partner = partner.reshape(ENK)

    xb = jnp.transpose(X[:, :, :3, :].reshape(B, 3 * N, 3), (0, 2, 1))
    xb8 = jnp.pad(xb, ((0, 0), (0, 5), (0, 0)))
    trig = _run_dih(xb8)
    dih = jnp.concatenate([trig[:, 0, :].reshape(B, N, 3),
                           trig[:, 1, :].reshape(B, N, 3)], axis=-1)
    dih8 = jnp.pad(dih, ((0, 0), (0, 0), (0, 2))).reshape(BN, 8)

    barange = jnp.arange(B, dtype=jnp.int32)[:, None, None]
    irange = jnp.arange(N, dtype=jnp.int32)[None, :, None]
    gidE = (barange * (N * N) + irange * N + E_idx).reshape(ENK)
    gidV = (barange * N + E_idx).reshape(ENK)
    dsel_f = dsel.reshape(ENK, 1)
    offs_f = offs.reshape(ENK, 1)

    (eg,) = _sc_multi_gather([(E_embed.reshape(B * N * N, HIN), gidE)])

    p = params
    row = lambda a: a.reshape(1, -1)
    he0 = _run_edgefeat(
        dsel_f, offs_f, eg,
        p['feat_edge']['w'], row(p['feat_edge']['b']),
        row(p['feat_edge_norm']['g']), row(p['feat_edge_norm']['b']),
        p['W_e']['w'][:H], p['W_e']['w'][H:], row(p['W_e']['b']))

    fnw = jnp.pad(p['feat_node']['w'], ((0, 2), (0, 0)))
    hv = _run_nodefeat(
        dih8, V_embed.reshape(BN, HIN), fnw, row(p['feat_node']['b']),
        row(p['feat_node_norm']['g']), row(p['feat_node_norm']['b']),
        p['W_v']['w'][:H], p['W_v']['w'][H:], row(p['W_v']['b']))

    he = he0
    for li in range(NLAYERS):
        lp = p['layers'][li]
        ep = lp['edge']
        npp = lp['node']
        if li == 0:
            hvj, hep = _sc_multi_gather([(hv, gidV), (he, partner)])
        else:
            (hvj,) = _sc_multi_gather([(hv, gidV)])
            hep = he
        w1 = ep['W1']['w']
        he_post = _run_edge_layer(
            he, hep, hvj,
            [w1[:H], w1[H:2 * H], w1[2 * H:], row(ep['W1']['b']),
             ep['W2']['w'], row(ep['W2']['b']),
             ep['W3']['w'], row(ep['W3']['b']),
             row(ep['norm']['g']), row(ep['norm']['b'])])
        (hepp,) = _sc_multi_gather([(he_post, partner)])
        nw1 = npp['W1']['w']
        nodep = [nw1[:H], nw1[H:2 * H], nw1[2 * H:], row(npp['W1']['b']),
                 npp['W2']['w'], row(npp['W2']['b']),
                 npp['W3']['w'], row(npp['W3']['b']),
                 row(npp['norm1']['g']), row(npp['norm1']['b']),
                 npp['ff_in']['w'], row(npp['ff_in']['b']),
                 npp['ff_out']['w'], row(npp['ff_out']['b']),
                 row(npp['norm2']['g']), row(npp['norm2']['b'])]
        if li < NLAYERS - 1:
            hv, he = _run_node_layer(hv, hvj, he_post, hepp, nodep)
        else:
            hv, etab = _run_node_layer_last(
                hv, hvj, he_post, hepp,
                nodep + [p['W_out']['w'], row(p['W_out']['b'])])

    return (etab.reshape(B, N, K, OUT), hv.reshape(B, N, H), E_idx)


# speedup vs baseline: 1.2158x; 1.2158x over previous
import functools

import jax
import jax.numpy as jnp
import numpy as np
from jax import lax
from jax.experimental import pallas as pl
from jax.experimental.pallas import tpu as pltpu
from jax.experimental.pallas import tpu_sc as plsc

B, N, H, HIN, K, OUT, NPE, NRBF, NLAYERS = 2, 512, 128, 128, 30, 400, 16, 16, 3
SCALE = 30.0
NK = N * K
ENK = B * NK
BN = B * N

def _dot(a, b):
    return jnp.dot(a, b, preferred_element_type=jnp.float32)


_PAR1 = pltpu.CompilerParams(dimension_semantics=("parallel",))


def _ln(x, g, b):
    mu = jnp.mean(x, axis=-1, keepdims=True)
    var = jnp.mean((x - mu) ** 2, axis=-1, keepdims=True)
    return (x - mu) / jnp.sqrt(var + 1e-5) * g + b


def _topk_body(xcol_ref, xrow_ref, chain_ref, eidx_ref, dsel_ref, offs_ref,
               partner_ref):
    xc = xcol_ref[0]
    xr = xrow_ref[0]
    ch = chain_ref[0]
    dx = xc[:, 0:1] - xr[0:1, :]
    dy = xc[:, 1:2] - xr[1:2, :]
    dz = xc[:, 2:3] - xr[2:3, :]
    s = (dx * dx + dy * dy) + dz * dz
    d = jnp.sqrt(s + 1e-6)
    lane = lax.broadcasted_iota(jnp.int32, (N, N), 1)
    cur = d
    idx_cols = []
    val_cols = []
    for _ in range(K):
        mn = jnp.min(cur, axis=1, keepdims=True)
        am = jnp.min(jnp.where(cur == mn, lane, N), axis=1, keepdims=True)
        idx_cols.append(am)
        val_cols.append(mn)
        cur = jnp.where(lane == am, jnp.float32(np.inf), cur)
    eidx = jnp.concatenate(idx_cols, axis=1)
    dsel = jnp.concatenate(val_cols, axis=1)
    t1 = jnp.sum((ch < 1).astype(jnp.int32), axis=1, keepdims=True)
    t2 = jnp.sum((ch < 2).astype(jnp.int32), axis=1, keepdims=True)
    t3 = jnp.sum((ch < 3).astype(jnp.int32), axis=1, keepdims=True)

    def chain_of(v):
        return ((v >= t1).astype(jnp.int32) + (v >= t2).astype(jnp.int32)
                + (v >= t3).astype(jnp.int32))

    i_col = lax.broadcasted_iota(jnp.int32, (N, 1), 0)
    same = (chain_of(eidx) == chain_of(i_col)).astype(jnp.float32)
    offs = (eidx - i_col).astype(jnp.float32) * same
    big = jnp.int32(9999)
    w = jnp.full((N, N), big, jnp.int32)
    for k in range(K):
        w = jnp.where(lane == idx_cols[k], jnp.int32(k), w)
    wt = w.T
    kp_cols = []
    for k in range(K):
        kp_cols.append(jnp.min(jnp.where(lane == idx_cols[k], wt, big),
                               axis=1, keepdims=True))
    kp = jnp.concatenate(kp_cols, axis=1)
    found = kp < big
    k_row = lax.broadcasted_iota(jnp.int32, (N, K), 1)
    b_off = pl.program_id(0) * NK
    partner = jnp.where(found, eidx * K + kp, i_col * K + k_row) + b_off
    eidx_ref[0] = eidx
    dsel_ref[0] = dsel
    offs_ref[0] = offs
    partner_ref[0] = partner


def _run_topk(xcol, xrow, chain3):
    return pl.pallas_call(
        _topk_body,
        grid=(B,),
        in_specs=[
            pl.BlockSpec((1, N, 8), lambda b: (b, 0, 0)),
            pl.BlockSpec((1, 8, N), lambda b: (b, 0, 0)),
            pl.BlockSpec((1, 1, N), lambda b: (b, 0, 0)),
        ],
        out_specs=[
            pl.BlockSpec((1, N, K), lambda b: (b, 0, 0)),
            pl.BlockSpec((1, N, K), lambda b: (b, 0, 0)),
            pl.BlockSpec((1, N, K), lambda b: (b, 0, 0)),
            pl.BlockSpec((1, N, K), lambda b: (b, 0, 0)),
        ],
        out_shape=[
            jax.ShapeDtypeStruct((B, N, K), jnp.int32),
            jax.ShapeDtypeStruct((B, N, K), jnp.float32),
            jax.ShapeDtypeStruct((B, N, K), jnp.float32),
            jax.ShapeDtypeStruct((B, N, K), jnp.int32),
        ],
        compiler_params=_PAR1,
    )(xcol, xrow, chain3)


def _dih_body(xb_ref, out_ref):
    xb = xb_ref[0]
    x = xb[0:1]
    y = xb[1:2]
    z = xb[2:3]

    def sh(a, s):
        return pltpu.roll(a, 3 * N - s, 1)

    def unit(ax, ay, az):
        s = (ax * ax + ay * ay) + az * az
        nrm = jnp.sqrt(s + 1e-12)
        return ax / nrm, ay / nrm, az / nrm

    def cross(ax, ay, az, bx, by, bz):
        return (ay * bz - az * by, az * bx - ax * bz, ax * by - ay * bx)

    dx, dy, dz = sh(x, 1) - x, sh(y, 1) - y, sh(z, 1) - z
    ux, uy, uz = unit(dx, dy, dz)
    vx, vy, vz = sh(ux, 1), sh(uy, 1), sh(uz, 1)
    wx, wy, wz = sh(ux, 2), sh(uy, 2), sh(uz, 2)
    ax_, ay_, az_ = cross(ux, uy, uz, vx, vy, vz)
    bx_, by_, bz_ = cross(vx, vy, vz, wx, wy, wz)
    n2x, n2y, n2z = unit(ax_, ay_, az_)
    n1x, n1y, n1z = unit(bx_, by_, bz_)
    cosd = (n2x * n1x + n2y * n1y) + n2z * n1z
    cosd = jnp.clip(cosd, -1.0 + 1e-7, 1.0 - 1e-7)
    sgn = jnp.sign((ux * n1x + uy * n1y) + uz * n1z)
    sind = sgn * jnp.sqrt(1.0 - cosd * cosd)
    cosf = pltpu.roll(cosd, 1, 1)
    sinf = pltpu.roll(sind, 1, 1)
    lt = lax.broadcasted_iota(jnp.int32, (1, 3 * N), 1)
    valid = (lt >= 1) & (lt <= 3 * N - 3)
    cosf = jnp.where(valid, cosf, 1.0)
    sinf = jnp.where(valid, sinf, 0.0)
    out_ref[0] = jnp.concatenate(
        [cosf, sinf, jnp.zeros((6, 3 * N), jnp.float32)], axis=0)


def _run_dih(xb8):
    return pl.pallas_call(
        _dih_body,
        grid=(B,),
        in_specs=[pl.BlockSpec((1, 8, 3 * N), lambda b: (b, 0, 0))],
        out_specs=pl.BlockSpec((1, 8, 3 * N), lambda b: (b, 0, 0)),
        out_shape=jax.ShapeDtypeStruct((B, 8, 3 * N), jnp.float32),
        compiler_params=_PAR1,
    )(xb8)


def _nodefeat_body(dih_ref, vemb_ref, fw_ref, fb_ref, g_ref, gb_ref,
                   wv1_ref, wv2_ref, bv_ref, out_ref):
    v = _ln(_dot(dih_ref[...], fw_ref[...]) + fb_ref[...],
            g_ref[...], gb_ref[...])
    out_ref[...] = (_dot(v, wv1_ref[...]) + _dot(vemb_ref[...], wv2_ref[...])
                    + bv_ref[...])


def _run_nodefeat(dih8, vemb, fw8, fb, g, gb, wv1, wv2, bv):
    return pl.pallas_call(
        _nodefeat_body,
        out_shape=jax.ShapeDtypeStruct((BN, H), jnp.float32),
    )(dih8, vemb, fw8, fb, g, gb, wv1, wv2, bv)


_RT2 = 2048

_SIGMA = 20.0 / NRBF


def _edgefeat_body(dsel_ref, offs_ref, eg_ref,
                   few_ref, feb_ref, g_ref, gb_ref,
                   we1_ref, we2_ref, be_ref, he_ref):
    dsel = dsel_ref[...]
    offs = offs_ref[...]
    io8 = lax.broadcasted_iota(jnp.int32, (1, NPE // 2), 1).astype(jnp.float32)
    freq = jnp.exp(io8 * jnp.float32(2.0 * (-np.log(10000.0) / NPE)))
    io16 = lax.broadcasted_iota(jnp.int32, (1, NRBF), 1).astype(jnp.float32)
    mu = 2.0 + io16 * jnp.float32(20.0 / (NRBF - 1))
    ang = offs * freq
    epos = jnp.concatenate([jnp.cos(ang), jnp.sin(ang)], axis=1)
    rbf = jnp.exp(-(((dsel - mu) / _SIGMA) ** 2))
    f = jnp.concatenate([epos, rbf], axis=1)
    e1 = _ln(_dot(f, few_ref[...]) + feb_ref[...], g_ref[...], gb_ref[...])
    he_ref[...] = (_dot(e1, we1_ref[...]) + _dot(eg_ref[...], we2_ref[...])
                   + be_ref[...])


def _run_edgefeat(dsel_f, offs_f, eg, few, feb, g, gb, we1, we2, be):
    nblk = ENK // _RT2
    full = lambda a: pl.BlockSpec(a.shape, lambda i: (0,) * a.ndim)
    return pl.pallas_call(
        _edgefeat_body,
        grid=(nblk,),
        in_specs=[
            pl.BlockSpec((_RT2, 1), lambda i: (i, 0)),
            pl.BlockSpec((_RT2, 1), lambda i: (i, 0)),
            pl.BlockSpec((_RT2, HIN), lambda i: (i, 0)),
            full(few), full(feb), full(g), full(gb),
            full(we1), full(we2), full(be),
        ],
        out_specs=pl.BlockSpec((_RT2, H), lambda i: (i, 0)),
        out_shape=jax.ShapeDtypeStruct((ENK, H), jnp.float32),
        compiler_params=_PAR1,
    )(dsel_f, offs_f, eg, few, feb, g, gb, we1, we2, be)


_NB = 128
_RE = _NB * K
_NGRID = BN // _NB


def _gelu(x):
    return jax.nn.gelu(x)


def _edge_layer_body(he_ref, hep_ref, hvj_ref,
                     w1a_ref, w1b_ref, w1c_ref, b1_ref,
                     w2_ref, b2_ref, w3_ref, b3_ref, g_ref, gb_ref, out_ref):
    hm = 0.5 * (he_ref[...] + hep_ref[...])
    hvj3 = hvj_ref[...].reshape(_NB, K, H)
    hvi = jnp.broadcast_to(hvj3[:, 0:1, :], (_NB, K, H)).reshape(_RE, H)
    x = (_dot(hvi, w1a_ref[...]) + _dot(hvj_ref[...], w1b_ref[...])
         + _dot(hm, w1c_ref[...]) + b1_ref[...])
    x = _gelu(x)
    x = _gelu(_dot(x, w2_ref[...]) + b2_ref[...])
    m = _dot(x, w3_ref[...]) + b3_ref[...]
    out_ref[...] = _ln(hm + m, g_ref[...], gb_ref[...])


def _run_edge_layer(he, hep, hvj, p):
    full = lambda a: pl.BlockSpec(a.shape, lambda i: (0,) * a.ndim)
    erow = pl.BlockSpec((_RE, H), lambda i: (i, 0))
    return pl.pallas_call(
        _edge_layer_body,
        grid=(_NGRID,),
        in_specs=[erow, erow, erow] + [full(a) for a in p],
        out_specs=erow,
        out_shape=jax.ShapeDtypeStruct((ENK, H), jnp.float32),
        compiler_params=_PAR1,
    )(he, hep, hvj, *p)


def _node_layer_body(hv_ref, hvj_ref, he_ref, hep_ref,
                     w1a_ref, w1b_ref, w1c_ref, b1_ref,
                     w2_ref, b2_ref, w3_ref, b3_ref,
                     g1_ref, g1b_ref, fi_ref, fib_ref, fo_ref, fob_ref,
                     g2_ref, g2b_ref, hv_out_ref, hm_out_ref):
    hm = 0.5 * (he_ref[...] + hep_ref[...])
    hm_out_ref[...] = hm
    hv = hv_ref[...]
    hv_e = jnp.broadcast_to(hv[:, None, :], (_NB, K, H)).reshape(_RE, H)
    x = (_dot(hv_e, w1a_ref[...]) + _dot(hvj_ref[...], w1b_ref[...])
         + _dot(hm, w1c_ref[...]) + b1_ref[...])
    x = _gelu(x)
    x = _gelu(_dot(x, w2_ref[...]) + b2_ref[...])
    m = _dot(x, w3_ref[...]) + b3_ref[...]
    dh = jnp.sum(m.reshape(_NB, K, H), axis=1) / SCALE
    hv1 = _ln(hv + dh, g1_ref[...], g1b_ref[...])
    ff = _dot(_gelu(_dot(hv1, fi_ref[...]) + fib_ref[...]), fo_ref[...]) \
        + fob_ref[...]
    hv_out_ref[...] = _ln(hv1 + ff, g2_ref[...], g2b_ref[...])


def _node_layer_last_body(hv_ref, hvj_ref, he_ref, hep_ref,
                          w1a_ref, w1b_ref, w1c_ref, b1_ref,
                          w2_ref, b2_ref, w3_ref, b3_ref,
                          g1_ref, g1b_ref, fi_ref, fib_ref, fo_ref, fob_ref,
                          g2_ref, g2b_ref, wo_ref, bo_ref,
                          hv_out_ref, etab_ref):
    hm = 0.5 * (he_ref[...] + hep_ref[...])
    etab_ref[...] = _dot(hm, wo_ref[...]) + bo_ref[...]
    hv = hv_ref[...]
    hv_e = jnp.broadcast_to(hv[:, None, :], (_NB, K, H)).reshape(_RE, H)
    x = (_dot(hv_e, w1a_ref[...]) + _dot(hvj_ref[...], w1b_ref[...])
         + _dot(hm, w1c_ref[...]) + b1_ref[...])
    x = _gelu(x)
    x = _gelu(_dot(x, w2_ref[...]) + b2_ref[...])
    m = _dot(x, w3_ref[...]) + b3_ref[...]
    dh = jnp.sum(m.reshape(_NB, K, H), axis=1) / SCALE
    hv1 = _ln(hv + dh, g1_ref[...], g1b_ref[...])
    ff = _dot(_gelu(_dot(hv1, fi_ref[...]) + fib_ref[...]), fo_ref[...]) \
        + fob_ref[...]
    hv_out_ref[...] = _ln(hv1 + ff, g2_ref[...], g2b_ref[...])


def _run_node_layer_last(hv, hvj, he, hep, p):
    full = lambda a: pl.BlockSpec(a.shape, lambda i: (0,) * a.ndim)
    erow = pl.BlockSpec((_RE, H), lambda i: (i, 0))
    nrow = pl.BlockSpec((_NB, H), lambda i: (i, 0))
    return pl.pallas_call(
        _node_layer_last_body,
        grid=(_NGRID,),
        in_specs=[nrow, erow, erow, erow] + [full(a) for a in p],
        out_specs=[nrow, pl.BlockSpec((_RE, OUT), lambda i: (i, 0))],
        out_shape=[
            jax.ShapeDtypeStruct((BN, H), jnp.float32),
            jax.ShapeDtypeStruct((ENK, OUT), jnp.float32),
        ],
        compiler_params=_PAR1,
    )(hv, hvj, he, hep, *p)


def _run_node_layer(hv, hvj, he, hep, p):
    full = lambda a: pl.BlockSpec(a.shape, lambda i: (0,) * a.ndim)
    erow = pl.BlockSpec((_RE, H), lambda i: (i, 0))
    nrow = pl.BlockSpec((_NB, H), lambda i: (i, 0))
    return pl.pallas_call(
        _node_layer_body,
        grid=(_NGRID,),
        in_specs=[nrow, erow, erow, erow] + [full(a) for a in p],
        out_specs=[nrow, erow],
        out_shape=[
            jax.ShapeDtypeStruct((BN, H), jnp.float32),
            jax.ShapeDtypeStruct((ENK, H), jnp.float32),
        ],
        compiler_params=_PAR1,
    )(hv, hvj, he, hep, *p)


_RT5 = 1920


def _etab_body(he_ref, w_ref, b_ref, out_ref):
    out_ref[...] = _dot(he_ref[...], w_ref[...]) + b_ref[...]


def _run_etab(he, w, b):
    return pl.pallas_call(
        _etab_body,
        grid=(ENK // _RT5,),
        in_specs=[
            pl.BlockSpec((_RT5, H), lambda i: (i, 0)),
            pl.BlockSpec((H, OUT), lambda i: (0, 0)),
            pl.BlockSpec((1, OUT), lambda i: (0, 0)),
        ],
        out_specs=pl.BlockSpec((_RT5, OUT), lambda i: (i, 0)),
        out_shape=jax.ShapeDtypeStruct((ENK, OUT), jnp.float32),
    )(he, w, b)


def _sc_multi_gather(specs):
    info = plsc.get_sparse_core_info()
    nc, ns = info.num_cores, info.num_subcores
    nw = nc * ns
    mesh = plsc.VectorSubcoreMesh(core_axis_name="c", subcore_axis_name="s")
    out_type = []
    scratch = []
    meta = []
    for table, idx in specs:
        v, d = table.shape
        r = idx.shape[0]
        assert r % (8 * nw) == 0, (r, nw)
        bpw = r // nw
        chunk = bpw if bpw <= 240 else 240
        assert bpw % chunk == 0
        out_type.append(jax.ShapeDtypeStruct((r, d), table.dtype))
        for _ in range(2):
            scratch.append(pltpu.VMEM((chunk,), jnp.int32))
            scratch.append(pltpu.VMEM((chunk, d), table.dtype))
            scratch.append(pltpu.SemaphoreType.DMA)
        meta.append((bpw, chunk))

    nspec = len(specs)

    @functools.partial(pl.kernel, out_type=tuple(out_type), mesh=mesh,
                       scratch_types=tuple(scratch))
    def gather_kernel(*refs):
        tables = refs[0:2 * nspec:2]
        idxs = refs[1:2 * nspec:2]
        outs = refs[2 * nspec:3 * nspec]
        scr = refs[3 * nspec:]
        wid = lax.axis_index("s") * nc + lax.axis_index("c")
        for si in range(nspec):
            bpw, chunk = meta[si]
            sl = scr[6 * si:6 * si + 6]
            idx_v = (sl[0], sl[3])
            rows_v = (sl[1], sl[4])
            sems = (sl[2], sl[5])
            base0 = wid * bpw
            nch = bpw // chunk

            def fire(c, si=si, base0=base0, chunk=chunk):
                base = base0 + c * chunk
                pltpu.sync_copy(idxs[si].at[pl.ds(base, chunk)],
                                idx_v[c % 2])
                return pltpu.async_copy(tables[si].at[idx_v[c % 2]],
                                        rows_v[c % 2], sems[c % 2])

            pend = {0: fire(0)}
            for c in range(nch):
                if c + 1 < nch:
                    pend[c + 1] = fire(c + 1)
                pend[c].wait()
                pltpu.sync_copy(
                    rows_v[c % 2],
                    outs[si].at[pl.ds(base0 + c * chunk, chunk)])

    flat_in = []
    for table, idx in specs:
        flat_in += [table, idx]
    res = gather_kernel(*flat_in)
    if not isinstance(res, (list, tuple)):
        res = (res,)
    return list(res)


def kernel(V_embed, E_embed, X, x_mask, chain_idx, params):
    Xca = X[:, :, 1, :]
    xcol = jnp.pad(Xca, ((0, 0), (0, 0), (0, 5)))
    xrow = jnp.transpose(xcol, (0, 2, 1))
    chain3 = chain_idx.astype(jnp.int32).reshape(B, 1, N)

    eidx, dsel, offs, partner = _run_topk(xcol, xrow, chain3)
    E_idx = eidx
    partner = partner.reshape(ENK)

    xb = jnp.transpose(X[:, :, :3, :].reshape(B, 3 * N, 3), (0, 2, 1))
    xb8 = jnp.pad(xb, ((0, 0), (0, 5), (0, 0)))
    trig = _run_dih(xb8)
    dih = jnp.concatenate([trig[:, 0, :].reshape(B, N, 3),
                           trig[:, 1, :].reshape(B, N, 3)], axis=-1)
    dih8 = jnp.pad(dih, ((0, 0), (0, 0), (0, 2))).reshape(BN, 8)

    barange = jnp.arange(B, dtype=jnp.int32)[:, None, None]
    irange = jnp.arange(N, dtype=jnp.int32)[None, :, None]
    gidE = (barange * (N * N) + irange * N + E_idx).reshape(ENK)
    gidV = (barange * N + E_idx).reshape(ENK)
    dsel_f = dsel.reshape(ENK, 1)
    offs_f = offs.reshape(ENK, 1)

    (eg,) = _sc_multi_gather([(E_embed.reshape(B * N * N, HIN), gidE)])

    p = params
    row = lambda a: a.reshape(1, -1)
    he0 = _run_edgefeat(
        dsel_f, offs_f, eg,
        p['feat_edge']['w'], row(p['feat_edge']['b']),
        row(p['feat_edge_norm']['g']), row(p['feat_edge_norm']['b']),
        p['W_e']['w'][:H], p['W_e']['w'][H:], row(p['W_e']['b']))

    fnw = jnp.pad(p['feat_node']['w'], ((0, 2), (0, 0)))
    hv = _run_nodefeat(
        dih8, V_embed.reshape(BN, HIN), fnw, row(p['feat_node']['b']),
        row(p['feat_node_norm']['g']), row(p['feat_node_norm']['b']),
        p['W_v']['w'][:H], p['W_v']['w'][H:], row(p['W_v']['b']))

    he = he0
    for li in range(NLAYERS):
        lp = p['layers'][li]
        ep = lp['edge']
        npp = lp['node']
        if li == 0:
            hvj, hep = _sc_multi_gather([(hv, gidV), (he, partner)])
        else:
            (hvj,) = _sc_multi_gather([(hv, gidV)])
            hep = he
        w1 = ep['W1']['w']
        he_post = _run_edge_layer(
            he, hep, hvj,
            [w1[:H], w1[H:2 * H], w1[2 * H:], row(ep['W1']['b']),
             ep['W2']['w'], row(ep['W2']['b']),
             ep['W3']['w'], row(ep['W3']['b']),
             row(ep['norm']['g']), row(ep['norm']['b'])])
        (hepp,) = _sc_multi_gather([(he_post, partner)])
        nw1 = npp['W1']['w']
        nodep = [nw1[:H], nw1[H:2 * H], nw1[2 * H:], row(npp['W1']['b']),
                 npp['W2']['w'], row(npp['W2']['b']),
                 npp['W3']['w'], row(npp['W3']['b']),
                 row(npp['norm1']['g']), row(npp['norm1']['b']),
                 npp['ff_in']['w'], row(npp['ff_in']['b']),
                 npp['ff_out']['w'], row(npp['ff_out']['b']),
                 row(npp['norm2']['g']), row(npp['norm2']['b'])]
        if li < NLAYERS - 1:
            hv, he = _run_node_layer(hv, hvj, he_post, hepp, nodep)
        else:
            hv, etab = _run_node_layer_last(
                hv, hvj, he_post, hepp,
                nodep + [p['W_out']['w'], row(p['W_out']['b'])])

    return (etab.reshape(B, N, K, OUT), hv.reshape(B, N, H), E_idx)
